# baseline (speedup 1.0000x reference)
import functools

import jax
import jax.numpy as jnp
from jax import lax
from jax.experimental import pallas as pl
from jax.experimental.pallas import tpu as pltpu
from jax.experimental.pallas import tpu_sc as plsc

NC = 2
NS = 16

HH = 128
CH_DEG = 40
CH_AGG = 80
NBUF = 5
NBUF_AGG = 4


def _sc_mesh():
    return plsc.VectorSubcoreMesh(core_axis_name="c", subcore_axis_name="s")


def _make_deg_kernel(n, k_chunks):
    rpt = n // NS

    @functools.partial(
        pl.kernel,
        out_type=jax.ShapeDtypeStruct((NC, n, 128), jnp.float32),
        mesh=_sc_mesh(),
        scratch_types=[
            pltpu.VMEM((k_chunks, CH_DEG), jnp.int32),
            pltpu.VMEM((CH_DEG, 128), jnp.float32),
            pltpu.VMEM_SHARED((n, 128), jnp.float32),
        ] + [pltpu.SemaphoreType.DMA] * NBUF,
    )
    def deg_kernel(col_hbm, ones_hbm, zeros_hbm, out_hbm, colv, onesv, accum,
                   *sems):
        c = lax.axis_index("c")
        s = lax.axis_index("s")
        pltpu.sync_copy(zeros_hbm.at[pl.ds(s * rpt, rpt)],
                        accum.at[pl.ds(s * rpt, rpt)])
        pltpu.sync_copy(ones_hbm, onesv)
        pltpu.sync_copy(col_hbm.at[c * NS + s], colv)
        plsc.subcore_barrier()

        for b in range(NBUF):
            pltpu.async_copy(onesv, accum.at[colv.at[b]], sems[b], add=True)

        def group(g, carry):
            for b in range(NBUF):
                k = g * NBUF + b
                pltpu.make_async_copy(onesv, accum.at[colv.at[k]],
                                      sems[b]).wait()

                @pl.when(k + NBUF < k_chunks)
                def _():
                    pltpu.async_copy(onesv, accum.at[colv.at[k + NBUF]],
                                     sems[b], add=True)
            return carry

        lax.fori_loop(0, k_chunks // NBUF, group, 0)
        plsc.subcore_barrier()
        pltpu.sync_copy(accum.at[pl.ds(s * rpt, rpt)],
                        out_hbm.at[c, pl.ds(s * rpt, rpt)])

    return deg_kernel


def _tc_xw_body(x_ref, w_ref, xw_ref):
    xw_ref[...] = jnp.dot(x_ref[...], w_ref[...],
                          preferred_element_type=jnp.float32)


def _tc_xw_call(x, w_gcn, n_pad, bn):
    d = x.shape[1]
    h0 = w_gcn.shape[1]
    grid = (n_pad + bn - 1) // bn
    return pl.pallas_call(
        _tc_xw_body,
        grid=(grid,),
        in_specs=[
            pl.BlockSpec((bn, d), lambda i: (i, 0)),
            pl.BlockSpec((d, h0), lambda i: (0, 0)),
        ],
        out_specs=pl.BlockSpec((bn, h0), lambda i: (i, 0)),
        out_shape=jax.ShapeDtypeStruct((n_pad, h0), jnp.float32),
    )(x, w_gcn)


def _tc_scale_body(xw_ref, pd_ref, y_ref, dinv_ref):
    xw = xw_ref[...]
    deg = 1.0 + (1.0 / 128.0) * jnp.sum(pd_ref[...], axis=(0, 2))
    dinv = lax.rsqrt(deg)
    dinv_ref[...] = dinv
    y = xw * dinv[:, None]
    y_ref[0] = y[:, :HH]
    y_ref[1] = y[:, HH:]


def _tc_scale_call(xw, pdeg, n_pad, bn):
    h0 = xw.shape[1]
    n = n_pad
    grid = (n + bn - 1) // bn
    return pl.pallas_call(
        _tc_scale_body,
        grid=(grid,),
        in_specs=[
            pl.BlockSpec((bn, h0), lambda i: (i, 0)),
            pl.BlockSpec((NC, bn, 128), lambda i: (0, i, 0)),
        ],
        out_specs=[
            pl.BlockSpec((NC, bn, HH), lambda i: (0, i, 0)),
            pl.BlockSpec((bn,), lambda i: (i,)),
        ],
        out_shape=[
            jax.ShapeDtypeStruct((NC, n, HH), jnp.float32),
            jax.ShapeDtypeStruct((n,), jnp.float32),
        ],
    )(xw, pdeg)


def _make_agg_kernel(n, k_chunks):
    rpt = n // NS

    NB = NBUF_AGG
    NR = 2 * NB

    @functools.partial(
        pl.kernel,
        out_type=jax.ShapeDtypeStruct((NC, n, HH), jnp.float32),
        mesh=_sc_mesh(),
        scratch_types=[
            pltpu.VMEM((NR, CH_AGG), jnp.int32),
            pltpu.VMEM((NB, CH_AGG), jnp.int32),
        ] + [pltpu.VMEM((CH_AGG, HH), jnp.float32)] * NB + [
            pltpu.VMEM_SHARED((n, HH), jnp.float32),
        ] + [pltpu.SemaphoreType.DMA] * (NR + 2 * NB),
    )
    def agg_kernel(row_hbm, col_hbm, y_hbm, out_hbm, rowv, colv, *rest):
        bufs = rest[:NB]
        accum = rest[NB]
        rsems = rest[NB + 1:NB + 1 + NR]
        csems = rest[NB + 1 + NR:NB + 1 + NR + NB]
        gsems = rest[NB + 1 + NR + NB:]
        c = lax.axis_index("c")
        s = lax.axis_index("s")
        pltpu.sync_copy(y_hbm.at[c, pl.ds(s * rpt, rpt)],
                        accum.at[pl.ds(s * rpt, rpt)])
        plsc.subcore_barrier()
        table = y_hbm.at[c]
        base = s * k_chunks

        def fire_ridx(k, j):
            pltpu.async_copy(row_hbm.at[base + k], rowv.at[j], rsems[j])

        def wait_ridx(k, j):
            pltpu.make_async_copy(row_hbm.at[base + k], rowv.at[j],
                                  rsems[j]).wait()

        def fire_cidx(k, b):
            pltpu.async_copy(col_hbm.at[base + k], colv.at[b], csems[b])

        def wait_cidx(k, b):
            pltpu.make_async_copy(col_hbm.at[base + k], colv.at[b],
                                  csems[b]).wait()

        def fire_gather(j, b):
            pltpu.async_copy(table.at[rowv.at[j]], bufs[b], gsems[b])

        def wait_gather(j, b):
            pltpu.make_async_copy(table.at[rowv.at[j]], bufs[b],
                                  gsems[b]).wait()

        for k in range(NR):
            fire_ridx(k, k)
        for k in range(NB):
            fire_cidx(k, k)
        for k in range(NB):
            wait_ridx(k, k)
            fire_gather(k, k)

        n_groups = k_chunks // NR

        def group(g, carry):
            for j in range(NR):
                b = j % NB
                j4 = (j + NB) % NR
                k = g * NR + j
                wait_gather(j, b)
                wait_cidx(k, b)
                pltpu.sync_copy(bufs[b], accum.at[colv.at[b]], add=True)

                @pl.when(k + NR < k_chunks)
                def _():
                    fire_ridx(k + NR, j)

                @pl.when(k + NB < k_chunks)
                def _():
                    wait_ridx(k + NB, j4)
                    fire_cidx(k + NB, b)
                    fire_gather(j4, b)
            return carry

        lax.fori_loop(0, n_groups, group, 0)
        for k in range(n_groups * NR, k_chunks):
            j = k % NR
            b = k % NB
            j4 = (j + NB) % NR
            wait_gather(j, b)
            wait_cidx(k, b)
            pltpu.sync_copy(bufs[b], accum.at[colv.at[b]], add=True)
            if k + NB < k_chunks:
                wait_ridx(k + NB, j4)
                fire_cidx(k + NB, b)
                fire_gather(j4, b)
        plsc.subcore_barrier()
        pltpu.sync_copy(accum.at[pl.ds(s * rpt, rpt)],
                        out_hbm.at[c, pl.ds(s * rpt, rpt)])

    return agg_kernel


def _tc_heads_body(h_ref, dinv_ref, bg_ref, wmu_ref, bmu_ref, wls_ref,
                   bls_ref, out_ref):
    hcat = jnp.concatenate([h_ref[0], h_ref[1]], axis=1)
    h = hcat * dinv_ref[...][:, None] + bg_ref[...][None, :]
    h = jnp.maximum(h, 0.0)
    mu = jnp.dot(h, wmu_ref[...], preferred_element_type=jnp.float32)
    mu = mu + bmu_ref[...][None, :]
    t = jnp.dot(h, wls_ref[...], preferred_element_type=jnp.float32)
    t = t + bls_ref[...][None, :]
    sigma = jnp.where(t > 0.0, t, jnp.exp(t) - 1.0) + (1.0 + 1e-14)
    out_ref[0] = mu
    out_ref[1] = sigma


def _tc_heads_call(h_agg, dinv, b_gcn, w_mu, b_mu, w_ls, b_ls, n, bn):
    h0 = b_gcn.shape[0]
    h1 = w_mu.shape[1]
    grid = (n + bn - 1) // bn
    return pl.pallas_call(
        _tc_heads_body,
        grid=(grid,),
        in_specs=[
            pl.BlockSpec((NC, bn, HH), lambda i: (0, i, 0)),
            pl.BlockSpec((bn,), lambda i: (i,)),
            pl.BlockSpec((h0,), lambda i: (0,)),
            pl.BlockSpec((h0, h1), lambda i: (0, 0)),
            pl.BlockSpec((h1,), lambda i: (0,)),
            pl.BlockSpec((h0, h1), lambda i: (0, 0)),
            pl.BlockSpec((h1,), lambda i: (0,)),
        ],
        out_specs=pl.BlockSpec((2, bn, h1), lambda i: (0, i, 0)),
        out_shape=jax.ShapeDtypeStruct((2, n, h1), jnp.float32),
    )(h_agg, dinv, b_gcn, w_mu, b_mu, w_ls, b_ls)


def kernel(x, edge_index, W_gcn, b_gcn, W_mu, b_mu, W_ls, b_ls):
    n, d = x.shape
    e = edge_index.shape[1]
    row = edge_index[0]
    col = edge_index[1]

    bn = 512
    n_pad = ((n + bn - 1) // bn) * bn

    kd = e // (NC * NS * CH_DEG)
    col_deg = col.reshape(NC * NS, kd, CH_DEG)
    ones8 = jnp.ones((CH_DEG, 128), jnp.float32)
    zeros8 = jnp.zeros((n_pad, 128), jnp.float32)
    xw = _tc_xw_call(x, W_gcn, n_pad, bn)
    pdeg = _make_deg_kernel(n_pad, kd)(col_deg, ones8, zeros8)

    y_split, dinv = _tc_scale_call(xw, pdeg, n_pad, bn)

    ka = e // (NS * CH_AGG)
    row3 = row.reshape(NS * ka, CH_AGG)
    col3 = col.reshape(NS * ka, CH_AGG)
    h_agg = _make_agg_kernel(n_pad, ka)(row3, col3, y_split)

    return _tc_heads_call(h_agg, dinv, b_gcn, W_mu, b_mu, W_ls, b_ls, n, bn)


# speedup vs baseline: 1.0052x; 1.0052x over previous
import functools

import jax
import jax.numpy as jnp
from jax import lax
from jax.experimental import pallas as pl
from jax.experimental.pallas import tpu as pltpu
from jax.experimental.pallas import tpu_sc as plsc

NC = 2
NS = 16

HH = 128
CH_DEG = 40
CH_AGG = 80
NBUF = 5
NBUF_AGG = 4


def _sc_mesh():
    return plsc.VectorSubcoreMesh(core_axis_name="c", subcore_axis_name="s")


def _make_deg_kernel(n, k_chunks):
    rpt = n // NS

    @functools.partial(
        pl.kernel,
        out_type=jax.ShapeDtypeStruct((NC, n, 128), jnp.float32),
        mesh=_sc_mesh(),
        scratch_types=[
            pltpu.VMEM((k_chunks, CH_DEG), jnp.int32),
            pltpu.VMEM((CH_DEG, 128), jnp.float32),
            pltpu.VMEM_SHARED((n, 128), jnp.float32),
        ] + [pltpu.SemaphoreType.DMA] * NBUF,
    )
    def deg_kernel(col_hbm, ones_hbm, zeros_hbm, out_hbm, colv, onesv, accum,
                   *sems):
        c = lax.axis_index("c")
        s = lax.axis_index("s")
        pltpu.sync_copy(zeros_hbm.at[pl.ds(s * rpt, rpt)],
                        accum.at[pl.ds(s * rpt, rpt)])
        pltpu.sync_copy(ones_hbm, onesv)
        pltpu.sync_copy(col_hbm.at[c * NS + s], colv)
        plsc.subcore_barrier()

        for b in range(NBUF):
            pltpu.async_copy(onesv, accum.at[colv.at[b]], sems[b], add=True)

        def group(g, carry):
            for b in range(NBUF):
                k = g * NBUF + b
                pltpu.make_async_copy(onesv, accum.at[colv.at[k]],
                                      sems[b]).wait()

                @pl.when(k + NBUF < k_chunks)
                def _():
                    pltpu.async_copy(onesv, accum.at[colv.at[k + NBUF]],
                                     sems[b], add=True)
            return carry

        lax.fori_loop(0, k_chunks // NBUF, group, 0)
        plsc.subcore_barrier()
        pltpu.sync_copy(accum.at[pl.ds(s * rpt, rpt)],
                        out_hbm.at[c, pl.ds(s * rpt, rpt)])

    return deg_kernel


def _tc_scale_body(x_ref, w_ref, pd_ref, y_ref, dinv_ref):
    xw = jnp.dot(x_ref[...], w_ref[...], preferred_element_type=jnp.float32)
    deg = 1.0 + (1.0 / 128.0) * jnp.sum(pd_ref[...], axis=(0, 2))
    dinv = lax.rsqrt(deg)
    dinv_ref[...] = dinv
    y = xw * dinv[:, None]
    y_ref[0] = y[:, :HH]
    y_ref[1] = y[:, HH:]


def _tc_scale_call(x, w_gcn, pdeg, n_pad, bn):
    d = x.shape[1]
    h0 = w_gcn.shape[1]
    n = n_pad
    grid = (n + bn - 1) // bn
    return pl.pallas_call(
        _tc_scale_body,
        grid=(grid,),
        in_specs=[
            pl.BlockSpec((bn, d), lambda i: (i, 0)),
            pl.BlockSpec((d, h0), lambda i: (0, 0)),
            pl.BlockSpec((NC, bn, 128), lambda i: (0, i, 0)),
        ],
        out_specs=[
            pl.BlockSpec((NC, bn, HH), lambda i: (0, i, 0)),
            pl.BlockSpec((bn,), lambda i: (i,)),
        ],
        out_shape=[
            jax.ShapeDtypeStruct((NC, n, HH), jnp.float32),
            jax.ShapeDtypeStruct((n,), jnp.float32),
        ],
    )(x, w_gcn, pdeg)


def _make_agg_kernel(n, k_chunks):
    rpt = n // NS

    NB = NBUF_AGG
    NR = 2 * NB

    @functools.partial(
        pl.kernel,
        out_type=jax.ShapeDtypeStruct((NC, n, HH), jnp.float32),
        mesh=_sc_mesh(),
        scratch_types=[
            pltpu.VMEM((NR, CH_AGG), jnp.int32),
            pltpu.VMEM((NB, CH_AGG), jnp.int32),
        ] + [pltpu.VMEM((CH_AGG, HH), jnp.float32)] * NB + [
            pltpu.VMEM_SHARED((n, HH), jnp.float32),
        ] + [pltpu.SemaphoreType.DMA] * (NR + 2 * NB),
    )
    def agg_kernel(row_hbm, col_hbm, y_hbm, out_hbm, rowv, colv, *rest):
        bufs = rest[:NB]
        accum = rest[NB]
        rsems = rest[NB + 1:NB + 1 + NR]
        csems = rest[NB + 1 + NR:NB + 1 + NR + NB]
        gsems = rest[NB + 1 + NR + NB:]
        c = lax.axis_index("c")
        s = lax.axis_index("s")
        pltpu.sync_copy(y_hbm.at[c, pl.ds(s * rpt, rpt)],
                        accum.at[pl.ds(s * rpt, rpt)])
        plsc.subcore_barrier()
        table = y_hbm.at[c]
        base = s * k_chunks

        def fire_ridx(k, j):
            pltpu.async_copy(row_hbm.at[base + k], rowv.at[j], rsems[j])

        def wait_ridx(k, j):
            pltpu.make_async_copy(row_hbm.at[base + k], rowv.at[j],
                                  rsems[j]).wait()

        def fire_cidx(k, b):
            pltpu.async_copy(col_hbm.at[base + k], colv.at[b], csems[b])

        def wait_cidx(k, b):
            pltpu.make_async_copy(col_hbm.at[base + k], colv.at[b],
                                  csems[b]).wait()

        def fire_gather(j, b):
            pltpu.async_copy(table.at[rowv.at[j]], bufs[b], gsems[b])

        def wait_gather(j, b):
            pltpu.make_async_copy(table.at[rowv.at[j]], bufs[b],
                                  gsems[b]).wait()

        for k in range(NR):
            fire_ridx(k, k)
        for k in range(NB):
            fire_cidx(k, k)
        for k in range(NB):
            wait_ridx(k, k)
            fire_gather(k, k)

        n_groups = k_chunks // NR

        def group(g, carry):
            for j in range(NR):
                b = j % NB
                j4 = (j + NB) % NR
                k = g * NR + j
                wait_gather(j, b)
                wait_cidx(k, b)
                pltpu.sync_copy(bufs[b], accum.at[colv.at[b]], add=True)

                @pl.when(k + NR < k_chunks)
                def _():
                    fire_ridx(k + NR, j)

                @pl.when(k + NB < k_chunks)
                def _():
                    wait_ridx(k + NB, j4)
                    fire_cidx(k + NB, b)
                    fire_gather(j4, b)
            return carry

        lax.fori_loop(0, n_groups, group, 0)
        for k in range(n_groups * NR, k_chunks):
            j = k % NR
            b = k % NB
            j4 = (j + NB) % NR
            wait_gather(j, b)
            wait_cidx(k, b)
            pltpu.sync_copy(bufs[b], accum.at[colv.at[b]], add=True)
            if k + NB < k_chunks:
                wait_ridx(k + NB, j4)
                fire_cidx(k + NB, b)
                fire_gather(j4, b)
        plsc.subcore_barrier()
        pltpu.sync_copy(accum.at[pl.ds(s * rpt, rpt)],
                        out_hbm.at[c, pl.ds(s * rpt, rpt)])

    return agg_kernel


def _tc_heads_body(h_ref, dinv_ref, bg_ref, wmu_ref, bmu_ref, wls_ref,
                   bls_ref, out_ref):
    hcat = jnp.concatenate([h_ref[0], h_ref[1]], axis=1)
    h = hcat * dinv_ref[...][:, None] + bg_ref[...][None, :]
    h = jnp.maximum(h, 0.0)
    mu = jnp.dot(h, wmu_ref[...], preferred_element_type=jnp.float32)
    mu = mu + bmu_ref[...][None, :]
    t = jnp.dot(h, wls_ref[...], preferred_element_type=jnp.float32)
    t = t + bls_ref[...][None, :]
    sigma = jnp.where(t > 0.0, t, jnp.exp(t) - 1.0) + (1.0 + 1e-14)
    out_ref[0] = mu
    out_ref[1] = sigma


def _tc_heads_call(h_agg, dinv, b_gcn, w_mu, b_mu, w_ls, b_ls, n, bn):
    h0 = b_gcn.shape[0]
    h1 = w_mu.shape[1]
    grid = (n + bn - 1) // bn
    return pl.pallas_call(
        _tc_heads_body,
        grid=(grid,),
        in_specs=[
            pl.BlockSpec((NC, bn, HH), lambda i: (0, i, 0)),
            pl.BlockSpec((bn,), lambda i: (i,)),
            pl.BlockSpec((h0,), lambda i: (0,)),
            pl.BlockSpec((h0, h1), lambda i: (0, 0)),
            pl.BlockSpec((h1,), lambda i: (0,)),
            pl.BlockSpec((h0, h1), lambda i: (0, 0)),
            pl.BlockSpec((h1,), lambda i: (0,)),
        ],
        out_specs=pl.BlockSpec((2, bn, h1), lambda i: (0, i, 0)),
        out_shape=jax.ShapeDtypeStruct((2, n, h1), jnp.float32),
    )(h_agg, dinv, b_gcn, w_mu, b_mu, w_ls, b_ls)


def kernel(x, edge_index, W_gcn, b_gcn, W_mu, b_mu, W_ls, b_ls):
    n, d = x.shape
    e = edge_index.shape[1]
    row = edge_index[0]
    col = edge_index[1]

    bn = 512
    n_pad = ((n + bn - 1) // bn) * bn

    kd = e // (NC * NS * CH_DEG)
    col_deg = col.reshape(NC * NS, kd, CH_DEG)
    ones8 = jnp.ones((CH_DEG, 128), jnp.float32)
    zeros8 = jnp.zeros((n_pad, 128), jnp.float32)
    pdeg = _make_deg_kernel(n_pad, kd)(col_deg, ones8, zeros8)

    y_split, dinv = _tc_scale_call(x, W_gcn, pdeg, n_pad, bn)

    ka = e // (NS * CH_AGG)
    row3 = row.reshape(NS * ka, CH_AGG)
    col3 = col.reshape(NS * ka, CH_AGG)
    h_agg = _make_agg_kernel(n_pad, ka)(row3, col3, y_split)

    return _tc_heads_call(h_agg, dinv, b_gcn, W_mu, b_mu, W_ls, b_ls, n, bn)


# speedup vs baseline: 1.0628x; 1.0572x over previous
import functools

import jax
import jax.numpy as jnp
from jax import lax
from jax.experimental import pallas as pl
from jax.experimental.pallas import tpu as pltpu
from jax.experimental.pallas import tpu_sc as plsc

NC = 2
NS = 16

HH = 128
CH_DEG = 40
CH_AGG = 80
NBUF = 5
NBUF_AGG = 4


def _sc_mesh():
    return plsc.VectorSubcoreMesh(core_axis_name="c", subcore_axis_name="s")


def _make_deg_kernel(n, k_chunks):
    rpt = n // NS

    @functools.partial(
        pl.kernel,
        out_type=jax.ShapeDtypeStruct((NC, n, 128), jnp.float32),
        mesh=_sc_mesh(),
        scratch_types=[
            pltpu.VMEM((k_chunks, CH_DEG), jnp.int32),
            pltpu.VMEM((CH_DEG, 128), jnp.float32),
            pltpu.VMEM_SHARED((n, 128), jnp.float32),
        ] + [pltpu.SemaphoreType.DMA] * NBUF,
    )
    def deg_kernel(col_hbm, ones_hbm, zeros_hbm, out_hbm, colv, onesv, accum,
                   *sems):
        c = lax.axis_index("c")
        s = lax.axis_index("s")
        pltpu.sync_copy(zeros_hbm.at[pl.ds(s * rpt, rpt)],
                        accum.at[pl.ds(s * rpt, rpt)])
        pltpu.sync_copy(ones_hbm, onesv)
        pltpu.sync_copy(col_hbm.at[c * NS + s], colv)
        plsc.subcore_barrier()

        for b in range(NBUF):
            pltpu.async_copy(onesv, accum.at[colv.at[b]], sems[b], add=True)

        def group(g, carry):
            for b in range(NBUF):
                k = g * NBUF + b
                pltpu.make_async_copy(onesv, accum.at[colv.at[k]],
                                      sems[b]).wait()

                @pl.when(k + NBUF < k_chunks)
                def _():
                    pltpu.async_copy(onesv, accum.at[colv.at[k + NBUF]],
                                     sems[b], add=True)
            return carry

        lax.fori_loop(0, k_chunks // NBUF, group, 0)
        plsc.subcore_barrier()
        pltpu.sync_copy(accum.at[pl.ds(s * rpt, rpt)],
                        out_hbm.at[c, pl.ds(s * rpt, rpt)])

    return deg_kernel


def _tc_scale_body(x_ref, w_ref, pd_ref, y_ref, dinv_ref):
    xw = jnp.dot(x_ref[...], w_ref[...], preferred_element_type=jnp.float32)
    deg = 1.0 + (1.0 / 128.0) * jnp.sum(pd_ref[...], axis=(0, 2))
    dinv = lax.rsqrt(deg)
    dinv_ref[...] = dinv
    y = xw * dinv[:, None]
    y_ref[0] = y[:, :HH]
    y_ref[1] = y[:, HH:]


def _tc_scale_call(x, w_gcn, pdeg, n_pad, bn):
    d = x.shape[1]
    h0 = w_gcn.shape[1]
    n = n_pad
    grid = (n + bn - 1) // bn
    return pl.pallas_call(
        _tc_scale_body,
        grid=(grid,),
        in_specs=[
            pl.BlockSpec((bn, d), lambda i: (i, 0)),
            pl.BlockSpec((d, h0), lambda i: (0, 0)),
            pl.BlockSpec((NC, bn, 128), lambda i: (0, i, 0)),
        ],
        out_specs=[
            pl.BlockSpec((NC, bn, HH), lambda i: (0, i, 0)),
            pl.BlockSpec((bn,), lambda i: (i,)),
        ],
        out_shape=[
            jax.ShapeDtypeStruct((NC, n, HH), jnp.float32),
            jax.ShapeDtypeStruct((n,), jnp.float32),
        ],
    )(x, w_gcn, pdeg)


def _make_agg_kernel(n, k_chunks):
    rpt = n // NS

    NB = NBUF_AGG
    NR = 2 * NB

    @functools.partial(
        pl.kernel,
        out_type=jax.ShapeDtypeStruct((NC, n, HH), jnp.float32),
        mesh=_sc_mesh(),
        scratch_types=[
            pltpu.VMEM((NR, CH_AGG), jnp.int32),
            pltpu.VMEM((NB, CH_AGG), jnp.int32),
        ] + [pltpu.VMEM((CH_AGG, HH), jnp.float32)] * NB + [
            pltpu.VMEM_SHARED((n, HH), jnp.float32),
        ] + [pltpu.SemaphoreType.DMA] * (NR + 2 * NB),
    )
    def agg_kernel(row_hbm, col_hbm, y_hbm, out_hbm, rowv, colv, *rest):
        bufs = rest[:NB]
        accum = rest[NB]
        rsems = rest[NB + 1:NB + 1 + NR]
        csems = rest[NB + 1 + NR:NB + 1 + NR + NB]
        gsems = rest[NB + 1 + NR + NB:]
        c = lax.axis_index("c")
        s = lax.axis_index("s")
        pltpu.sync_copy(y_hbm.at[c, pl.ds(s * rpt, rpt)],
                        accum.at[pl.ds(s * rpt, rpt)])
        plsc.subcore_barrier()
        table = y_hbm.at[c]
        base = s * k_chunks

        def fire_ridx(k, j):
            pltpu.async_copy(row_hbm.at[base + k], rowv.at[j], rsems[j])

        def wait_ridx(k, j):
            pltpu.make_async_copy(row_hbm.at[base + k], rowv.at[j],
                                  rsems[j]).wait()

        def fire_cidx(k, b):
            pltpu.async_copy(col_hbm.at[base + k], colv.at[b], csems[b])

        def wait_cidx(k, b):
            pltpu.make_async_copy(col_hbm.at[base + k], colv.at[b],
                                  csems[b]).wait()

        def fire_gather(j, b):
            pltpu.async_copy(table.at[rowv.at[j]], bufs[b], gsems[b])

        def wait_gather(j, b):
            pltpu.make_async_copy(table.at[rowv.at[j]], bufs[b],
                                  gsems[b]).wait()

        for k in range(NR):
            fire_ridx(k, k)
        for k in range(NB):
            fire_cidx(k, k)
        for k in range(NB):
            wait_ridx(k, k)
            fire_gather(k, k)

        n_groups = k_chunks // NR

        def group(g, carry):
            for j in range(NR):
                b = j % NB
                j4 = (j + NB) % NR
                k = g * NR + j
                wait_gather(j, b)
                wait_cidx(k, b)
                pltpu.sync_copy(bufs[b], accum.at[colv.at[b]], add=True)

                @pl.when(k + NR < k_chunks)
                def _():
                    fire_ridx(k + NR, j)

                @pl.when(k + NB < k_chunks)
                def _():
                    wait_ridx(k + NB, j4)
                    fire_cidx(k + NB, b)
                    fire_gather(j4, b)
            return carry

        lax.fori_loop(0, n_groups, group, 0)
        for k in range(n_groups * NR, k_chunks):
            j = k % NR
            b = k % NB
            j4 = (j + NB) % NR
            wait_gather(j, b)
            wait_cidx(k, b)
            pltpu.sync_copy(bufs[b], accum.at[colv.at[b]], add=True)
            if k + NB < k_chunks:
                wait_ridx(k + NB, j4)
                fire_cidx(k + NB, b)
                fire_gather(j4, b)
        plsc.subcore_barrier()
        pltpu.sync_copy(accum.at[pl.ds(s * rpt, rpt)],
                        out_hbm.at[c, pl.ds(s * rpt, rpt)])

    return agg_kernel


def _tc_heads_body(h_ref, dinv_ref, bg_ref, wmu_ref, bmu_ref, wls_ref,
                   bls_ref, out_ref):
    hcat = jnp.concatenate([h_ref[0], h_ref[1]], axis=1)
    h = hcat * dinv_ref[...][:, None] + bg_ref[...][None, :]
    h = jnp.maximum(h, 0.0)
    mu = jnp.dot(h, wmu_ref[...], preferred_element_type=jnp.float32)
    mu = mu + bmu_ref[...][None, :]
    t = jnp.dot(h, wls_ref[...], preferred_element_type=jnp.float32)
    t = t + bls_ref[...][None, :]
    sigma = jnp.where(t > 0.0, t, jnp.exp(t) - 1.0) + (1.0 + 1e-14)
    out_ref[0] = mu
    out_ref[1] = sigma


def _tc_heads_call(h_agg, dinv, b_gcn, w_mu, b_mu, w_ls, b_ls, n, bn):
    h0 = b_gcn.shape[0]
    h1 = w_mu.shape[1]
    grid = (n + bn - 1) // bn
    return pl.pallas_call(
        _tc_heads_body,
        grid=(grid,),
        in_specs=[
            pl.BlockSpec((NC, bn, HH), lambda i: (0, i, 0)),
            pl.BlockSpec((bn,), lambda i: (i,)),
            pl.BlockSpec((h0,), lambda i: (0,)),
            pl.BlockSpec((h0, h1), lambda i: (0, 0)),
            pl.BlockSpec((h1,), lambda i: (0,)),
            pl.BlockSpec((h0, h1), lambda i: (0, 0)),
            pl.BlockSpec((h1,), lambda i: (0,)),
        ],
        out_specs=pl.BlockSpec((2, bn, h1), lambda i: (0, i, 0)),
        out_shape=jax.ShapeDtypeStruct((2, n, h1), jnp.float32),
    )(h_agg, dinv, b_gcn, w_mu, b_mu, w_ls, b_ls)


def kernel(x, edge_index, W_gcn, b_gcn, W_mu, b_mu, W_ls, b_ls):
    n, d = x.shape
    e = edge_index.shape[1]
    row = edge_index[0]
    col = edge_index[1]

    bn = 1024
    n_pad = ((n + bn - 1) // bn) * bn

    kd = e // (NC * NS * CH_DEG)
    col_deg = col.reshape(NC * NS, kd, CH_DEG)
    ones8 = jnp.ones((CH_DEG, 128), jnp.float32)
    zeros8 = jnp.zeros((n_pad, 128), jnp.float32)
    pdeg = _make_deg_kernel(n_pad, kd)(col_deg, ones8, zeros8)

    y_split, dinv = _tc_scale_call(x, W_gcn, pdeg, n_pad, bn)

    ka = e // (NS * CH_AGG)
    row3 = row.reshape(NS * ka, CH_AGG)
    col3 = col.reshape(NS * ka, CH_AGG)
    h_agg = _make_agg_kernel(n_pad, ka)(row3, col3, y_split)

    return _tc_heads_call(h_agg, dinv, b_gcn, W_mu, b_mu, W_ls, b_ls, n, bn)


# speedup vs baseline: 1.0859x; 1.0217x over previous
import functools

import jax
import jax.numpy as jnp
from jax import lax
from jax.experimental import pallas as pl
from jax.experimental.pallas import tpu as pltpu
from jax.experimental.pallas import tpu_sc as plsc

NC = 2
NS = 16

HH = 128
CH_DEG = 40
CH_AGG = 80
NBUF = 5
NBUF_AGG = 4


def _sc_mesh():
    return plsc.VectorSubcoreMesh(core_axis_name="c", subcore_axis_name="s")


def _make_deg_kernel(n, k_chunks):
    rpt = n // NS

    @functools.partial(
        pl.kernel,
        out_type=jax.ShapeDtypeStruct((NC, n, 128), jnp.float32),
        mesh=_sc_mesh(),
        scratch_types=[
            pltpu.VMEM((k_chunks, CH_DEG), jnp.int32),
            pltpu.VMEM((CH_DEG, 128), jnp.float32),
            pltpu.VMEM_SHARED((n, 128), jnp.float32),
        ] + [pltpu.SemaphoreType.DMA] * NBUF,
    )
    def deg_kernel(col_hbm, ones_hbm, zeros_hbm, out_hbm, colv, onesv, accum,
                   *sems):
        c = lax.axis_index("c")
        s = lax.axis_index("s")
        pltpu.sync_copy(zeros_hbm.at[pl.ds(s * rpt, rpt)],
                        accum.at[pl.ds(s * rpt, rpt)])
        pltpu.sync_copy(ones_hbm, onesv)
        pltpu.sync_copy(col_hbm.at[c * NS + s], colv)
        plsc.subcore_barrier()

        for b in range(NBUF):
            pltpu.async_copy(onesv, accum.at[colv.at[b]], sems[b], add=True)

        def group(g, carry):
            for b in range(NBUF):
                k = g * NBUF + b
                pltpu.make_async_copy(onesv, accum.at[colv.at[k]],
                                      sems[b]).wait()

                @pl.when(k + NBUF < k_chunks)
                def _():
                    pltpu.async_copy(onesv, accum.at[colv.at[k + NBUF]],
                                     sems[b], add=True)
            return carry

        lax.fori_loop(0, k_chunks // NBUF, group, 0)
        plsc.subcore_barrier()
        pltpu.sync_copy(accum.at[pl.ds(s * rpt, rpt)],
                        out_hbm.at[c, pl.ds(s * rpt, rpt)])

    return deg_kernel


def _tc_scale_body(x_ref, w_ref, pd_ref, y_ref, dinv_ref):
    xw = jnp.dot(x_ref[...], w_ref[...], preferred_element_type=jnp.float32)
    deg = 1.0 + (1.0 / 128.0) * jnp.sum(pd_ref[...], axis=(0, 2))
    dinv = lax.rsqrt(deg)
    dinv_ref[...] = dinv
    y = xw * dinv[:, None]
    y_ref[0] = y[:, :HH]
    y_ref[1] = y[:, HH:]


def _tc_scale_call(x, w_gcn, pdeg, n_pad, bn):
    d = x.shape[1]
    h0 = w_gcn.shape[1]
    n = n_pad
    grid = (n + bn - 1) // bn
    return pl.pallas_call(
        _tc_scale_body,
        grid=(grid,),
        in_specs=[
            pl.BlockSpec((bn, d), lambda i: (i, 0)),
            pl.BlockSpec((d, h0), lambda i: (0, 0)),
            pl.BlockSpec((NC, bn, 128), lambda i: (0, i, 0)),
        ],
        out_specs=[
            pl.BlockSpec((NC, bn, HH), lambda i: (0, i, 0)),
            pl.BlockSpec((bn,), lambda i: (i,)),
        ],
        out_shape=[
            jax.ShapeDtypeStruct((NC, n, HH), jnp.float32),
            jax.ShapeDtypeStruct((n,), jnp.float32),
        ],
    )(x, w_gcn, pdeg)


def _make_agg_kernel(n, k_chunks):
    rpt = n // NS

    NB = NBUF_AGG
    NR = 2 * NB

    @functools.partial(
        pl.kernel,
        out_type=jax.ShapeDtypeStruct((NC, n, HH), jnp.float32),
        mesh=_sc_mesh(),
        scratch_types=[
            pltpu.VMEM((NR, CH_AGG), jnp.int32),
            pltpu.VMEM((NB, CH_AGG), jnp.int32),
        ] + [pltpu.VMEM((CH_AGG, HH), jnp.float32)] * NB + [
            pltpu.VMEM_SHARED((n, HH), jnp.float32),
        ] + [pltpu.SemaphoreType.DMA] * (NR + 2 * NB),
    )
    def agg_kernel(row_hbm, col_hbm, y_hbm, out_hbm, rowv, colv, *rest):
        bufs = rest[:NB]
        accum = rest[NB]
        rsems = rest[NB + 1:NB + 1 + NR]
        csems = rest[NB + 1 + NR:NB + 1 + NR + NB]
        gsems = rest[NB + 1 + NR + NB:]
        c = lax.axis_index("c")
        s = lax.axis_index("s")
        pltpu.sync_copy(y_hbm.at[c, pl.ds(s * rpt, rpt)],
                        accum.at[pl.ds(s * rpt, rpt)])
        plsc.subcore_barrier()
        table = y_hbm.at[c]
        base = s * k_chunks

        def fire_ridx(k, j):
            pltpu.async_copy(row_hbm.at[base + k], rowv.at[j], rsems[j])

        def wait_ridx(k, j):
            pltpu.make_async_copy(row_hbm.at[base + k], rowv.at[j],
                                  rsems[j]).wait()

        def fire_cidx(k, b):
            pltpu.async_copy(col_hbm.at[base + k], colv.at[b], csems[b])

        def wait_cidx(k, b):
            pltpu.make_async_copy(col_hbm.at[base + k], colv.at[b],
                                  csems[b]).wait()

        def fire_gather(j, b):
            pltpu.async_copy(table.at[rowv.at[j]], bufs[b], gsems[b])

        def wait_gather(j, b):
            pltpu.make_async_copy(table.at[rowv.at[j]], bufs[b],
                                  gsems[b]).wait()

        for k in range(NR):
            fire_ridx(k, k)
        for k in range(NB):
            fire_cidx(k, k)
        for k in range(NB):
            wait_ridx(k, k)
            fire_gather(k, k)

        n_groups = k_chunks // NR

        def group(g, carry):
            for j in range(NR):
                b = j % NB
                j4 = (j + NB) % NR
                k = g * NR + j
                wait_gather(j, b)
                wait_cidx(k, b)
                pltpu.sync_copy(bufs[b], accum.at[colv.at[b]], add=True)

                @pl.when(k + NR < k_chunks)
                def _():
                    fire_ridx(k + NR, j)

                @pl.when(k + NB < k_chunks)
                def _():
                    wait_ridx(k + NB, j4)
                    fire_cidx(k + NB, b)
                    fire_gather(j4, b)
            return carry

        lax.fori_loop(0, n_groups, group, 0)
        for k in range(n_groups * NR, k_chunks):
            j = k % NR
            b = k % NB
            j4 = (j + NB) % NR
            wait_gather(j, b)
            wait_cidx(k, b)
            pltpu.sync_copy(bufs[b], accum.at[colv.at[b]], add=True)
            if k + NB < k_chunks:
                wait_ridx(k + NB, j4)
                fire_cidx(k + NB, b)
                fire_gather(j4, b)
        plsc.subcore_barrier()
        pltpu.sync_copy(accum.at[pl.ds(s * rpt, rpt)],
                        out_hbm.at[c, pl.ds(s * rpt, rpt)])

    return agg_kernel


def _tc_heads_body(h_ref, dinv_ref, bg_ref, wmu_ref, bmu_ref, wls_ref,
                   bls_ref, out_ref):
    hcat = jnp.concatenate([h_ref[0], h_ref[1]], axis=1)
    h = hcat * dinv_ref[...][:, None] + bg_ref[...][None, :]
    h = jnp.maximum(h, 0.0)
    mu = jnp.dot(h, wmu_ref[...], preferred_element_type=jnp.float32)
    mu = mu + bmu_ref[...][None, :]
    t = jnp.dot(h, wls_ref[...], preferred_element_type=jnp.float32)
    t = t + bls_ref[...][None, :]
    sigma = jnp.where(t > 0.0, t, jnp.exp(t) - 1.0) + (1.0 + 1e-14)
    out_ref[0] = mu
    out_ref[1] = sigma


def _tc_heads_call(h_agg, dinv, b_gcn, w_mu, b_mu, w_ls, b_ls, n, bn):
    h0 = b_gcn.shape[0]
    h1 = w_mu.shape[1]
    grid = (n + bn - 1) // bn
    return pl.pallas_call(
        _tc_heads_body,
        grid=(grid,),
        in_specs=[
            pl.BlockSpec((NC, bn, HH), lambda i: (0, i, 0)),
            pl.BlockSpec((bn,), lambda i: (i,)),
            pl.BlockSpec((h0,), lambda i: (0,)),
            pl.BlockSpec((h0, h1), lambda i: (0, 0)),
            pl.BlockSpec((h1,), lambda i: (0,)),
            pl.BlockSpec((h0, h1), lambda i: (0, 0)),
            pl.BlockSpec((h1,), lambda i: (0,)),
        ],
        out_specs=pl.BlockSpec((2, bn, h1), lambda i: (0, i, 0)),
        out_shape=jax.ShapeDtypeStruct((2, n, h1), jnp.float32),
    )(h_agg, dinv, b_gcn, w_mu, b_mu, w_ls, b_ls)


def kernel(x, edge_index, W_gcn, b_gcn, W_mu, b_mu, W_ls, b_ls):
    n, d = x.shape
    e = edge_index.shape[1]
    row = edge_index[0]
    col = edge_index[1]

    bn = 2048
    n_pad = ((n + bn - 1) // bn) * bn

    kd = e // (NC * NS * CH_DEG)
    col_deg = col.reshape(NC * NS, kd, CH_DEG)
    ones8 = jnp.ones((CH_DEG, 128), jnp.float32)
    zeros8 = jnp.zeros((n_pad, 128), jnp.float32)
    pdeg = _make_deg_kernel(n_pad, kd)(col_deg, ones8, zeros8)

    y_split, dinv = _tc_scale_call(x, W_gcn, pdeg, n_pad, bn)

    ka = e // (NS * CH_AGG)
    row3 = row.reshape(NS * ka, CH_AGG)
    col3 = col.reshape(NS * ka, CH_AGG)
    h_agg = _make_agg_kernel(n_pad, ka)(row3, col3, y_split)

    return _tc_heads_call(h_agg, dinv, b_gcn, W_mu, b_mu, W_ls, b_ls, n, bn)


# speedup vs baseline: 1.0945x; 1.0080x over previous
import functools

import jax
import jax.numpy as jnp
from jax import lax
from jax.experimental import pallas as pl
from jax.experimental.pallas import tpu as pltpu
from jax.experimental.pallas import tpu_sc as plsc

NC = 2
NS = 16

HH = 128
CH_DEG = 40
CH_AGG = 80
NBUF = 5
NBUF_AGG = 4


def _sc_mesh():
    return plsc.VectorSubcoreMesh(core_axis_name="c", subcore_axis_name="s")


def _make_deg_kernel(n, k_chunks):
    rpt = n // NS

    @functools.partial(
        pl.kernel,
        out_type=jax.ShapeDtypeStruct((NC, n, 128), jnp.float32),
        mesh=_sc_mesh(),
        scratch_types=[
            pltpu.VMEM((k_chunks, CH_DEG), jnp.int32),
            pltpu.VMEM((CH_DEG, 128), jnp.float32),
            pltpu.VMEM_SHARED((n, 128), jnp.float32),
        ] + [pltpu.SemaphoreType.DMA] * NBUF,
    )
    def deg_kernel(col_hbm, ones_hbm, zeros_hbm, out_hbm, colv, onesv, accum,
                   *sems):
        c = lax.axis_index("c")
        s = lax.axis_index("s")
        pltpu.sync_copy(zeros_hbm.at[pl.ds(s * rpt, rpt)],
                        accum.at[pl.ds(s * rpt, rpt)])
        pltpu.sync_copy(ones_hbm, onesv)
        pltpu.sync_copy(col_hbm.at[c * NS + s], colv)
        plsc.subcore_barrier()

        for b in range(NBUF):
            pltpu.async_copy(onesv, accum.at[colv.at[b]], sems[b], add=True)

        def group(g, carry):
            for b in range(NBUF):
                k = g * NBUF + b
                pltpu.make_async_copy(onesv, accum.at[colv.at[k]],
                                      sems[b]).wait()

                @pl.when(k + NBUF < k_chunks)
                def _():
                    pltpu.async_copy(onesv, accum.at[colv.at[k + NBUF]],
                                     sems[b], add=True)
            return carry

        lax.fori_loop(0, k_chunks // NBUF, group, 0)
        plsc.subcore_barrier()
        pltpu.sync_copy(accum.at[pl.ds(s * rpt, rpt)],
                        out_hbm.at[c, pl.ds(s * rpt, rpt)])

    return deg_kernel


def _tc_scale_body(x_ref, w_ref, pd_ref, y_ref, dinv_ref):
    xw = jnp.dot(x_ref[...], w_ref[...], preferred_element_type=jnp.float32)
    deg = 1.0 + (1.0 / 128.0) * jnp.sum(pd_ref[...], axis=(0, 2))
    dinv = lax.rsqrt(deg)
    dinv_ref[...] = dinv
    y = xw * dinv[:, None]
    y_ref[0] = y[:, :HH]
    y_ref[1] = y[:, HH:]


def _tc_scale_call(x, w_gcn, pdeg, n_pad, bn):
    d = x.shape[1]
    h0 = w_gcn.shape[1]
    n = n_pad
    grid = (n + bn - 1) // bn
    return pl.pallas_call(
        _tc_scale_body,
        grid=(grid,),
        in_specs=[
            pl.BlockSpec((bn, d), lambda i: (i, 0)),
            pl.BlockSpec((d, h0), lambda i: (0, 0)),
            pl.BlockSpec((NC, bn, 128), lambda i: (0, i, 0)),
        ],
        out_specs=[
            pl.BlockSpec((NC, bn, HH), lambda i: (0, i, 0)),
            pl.BlockSpec((bn,), lambda i: (i,)),
        ],
        out_shape=[
            jax.ShapeDtypeStruct((NC, n, HH), jnp.float32),
            jax.ShapeDtypeStruct((n,), jnp.float32),
        ],
    )(x, w_gcn, pdeg)


def _make_agg_kernel(n, k_chunks):
    rpt = n // NS

    NB = NBUF_AGG
    NR = 2 * NB

    @functools.partial(
        pl.kernel,
        out_type=jax.ShapeDtypeStruct((NC, n, HH), jnp.float32),
        mesh=_sc_mesh(),
        scratch_types=[
            pltpu.VMEM((NR, CH_AGG), jnp.int32),
            pltpu.VMEM((NB, CH_AGG), jnp.int32),
        ] + [pltpu.VMEM((CH_AGG, HH), jnp.float32)] * NB + [
            pltpu.VMEM_SHARED((n, HH), jnp.float32),
        ] + [pltpu.SemaphoreType.DMA] * (NR + 2 * NB),
    )
    def agg_kernel(row_hbm, col_hbm, y_hbm, out_hbm, rowv, colv, *rest):
        bufs = rest[:NB]
        accum = rest[NB]
        rsems = rest[NB + 1:NB + 1 + NR]
        csems = rest[NB + 1 + NR:NB + 1 + NR + NB]
        gsems = rest[NB + 1 + NR + NB:]
        c = lax.axis_index("c")
        s = lax.axis_index("s")
        pltpu.sync_copy(y_hbm.at[c, pl.ds(s * rpt, rpt)],
                        accum.at[pl.ds(s * rpt, rpt)])
        plsc.subcore_barrier()
        table = y_hbm.at[c]
        base = s * k_chunks

        def fire_ridx(k, j):
            pltpu.async_copy(row_hbm.at[base + k], rowv.at[j], rsems[j])

        def wait_ridx(k, j):
            pltpu.make_async_copy(row_hbm.at[base + k], rowv.at[j],
                                  rsems[j]).wait()

        def fire_cidx(k, b):
            pltpu.async_copy(col_hbm.at[base + k], colv.at[b], csems[b])

        def wait_cidx(k, b):
            pltpu.make_async_copy(col_hbm.at[base + k], colv.at[b],
                                  csems[b]).wait()

        def fire_gather(j, b):
            pltpu.async_copy(table.at[rowv.at[j]], bufs[b], gsems[b])

        def wait_gather(j, b):
            pltpu.make_async_copy(table.at[rowv.at[j]], bufs[b],
                                  gsems[b]).wait()

        for k in range(NR):
            fire_ridx(k, k)
        for k in range(NB):
            fire_cidx(k, k)
        for k in range(NB):
            wait_ridx(k, k)
            fire_gather(k, k)

        n_groups = k_chunks // NR

        def group(g, carry):
            for j in range(NR):
                b = j % NB
                j4 = (j + NB) % NR
                k = g * NR + j
                wait_gather(j, b)
                wait_cidx(k, b)
                pltpu.sync_copy(bufs[b], accum.at[colv.at[b]], add=True)

                @pl.when(k + NR < k_chunks)
                def _():
                    fire_ridx(k + NR, j)

                @pl.when(k + NB < k_chunks)
                def _():
                    wait_ridx(k + NB, j4)
                    fire_cidx(k + NB, b)
                    fire_gather(j4, b)
            return carry

        lax.fori_loop(0, n_groups, group, 0)
        for k in range(n_groups * NR, k_chunks):
            j = k % NR
            b = k % NB
            j4 = (j + NB) % NR
            wait_gather(j, b)
            wait_cidx(k, b)
            pltpu.sync_copy(bufs[b], accum.at[colv.at[b]], add=True)
            if k + NB < k_chunks:
                wait_ridx(k + NB, j4)
                fire_cidx(k + NB, b)
                fire_gather(j4, b)
        plsc.subcore_barrier()
        pltpu.sync_copy(accum.at[pl.ds(s * rpt, rpt)],
                        out_hbm.at[c, pl.ds(s * rpt, rpt)])

    return agg_kernel


def _tc_heads_body(h_ref, dinv_ref, bg_ref, wmu_ref, bmu_ref, wls_ref,
                   bls_ref, out_ref):
    hcat = jnp.concatenate([h_ref[0], h_ref[1]], axis=1)
    h = hcat * dinv_ref[...][:, None] + bg_ref[...][None, :]
    h = jnp.maximum(h, 0.0)
    mu = jnp.dot(h, wmu_ref[...], preferred_element_type=jnp.float32)
    mu = mu + bmu_ref[...][None, :]
    t = jnp.dot(h, wls_ref[...], preferred_element_type=jnp.float32)
    t = t + bls_ref[...][None, :]
    sigma = jnp.where(t > 0.0, t, jnp.exp(t) - 1.0) + (1.0 + 1e-14)
    out_ref[0] = mu
    out_ref[1] = sigma


def _tc_heads_call(h_agg, dinv, b_gcn, w_mu, b_mu, w_ls, b_ls, n, bn):
    h0 = b_gcn.shape[0]
    h1 = w_mu.shape[1]
    grid = (n + bn - 1) // bn
    return pl.pallas_call(
        _tc_heads_body,
        grid=(grid,),
        in_specs=[
            pl.BlockSpec((NC, bn, HH), lambda i: (0, i, 0)),
            pl.BlockSpec((bn,), lambda i: (i,)),
            pl.BlockSpec((h0,), lambda i: (0,)),
            pl.BlockSpec((h0, h1), lambda i: (0, 0)),
            pl.BlockSpec((h1,), lambda i: (0,)),
            pl.BlockSpec((h0, h1), lambda i: (0, 0)),
            pl.BlockSpec((h1,), lambda i: (0,)),
        ],
        out_specs=pl.BlockSpec((2, bn, h1), lambda i: (0, i, 0)),
        out_shape=jax.ShapeDtypeStruct((2, n, h1), jnp.float32),
    )(h_agg, dinv, b_gcn, w_mu, b_mu, w_ls, b_ls)


def kernel(x, edge_index, W_gcn, b_gcn, W_mu, b_mu, W_ls, b_ls):
    n, d = x.shape
    e = edge_index.shape[1]
    row = edge_index[0]
    col = edge_index[1]

    bn = 10240
    n_pad = ((n + bn - 1) // bn) * bn

    kd = e // (NC * NS * CH_DEG)
    col_deg = col.reshape(NC * NS, kd, CH_DEG)
    ones8 = jnp.ones((CH_DEG, 128), jnp.float32)
    zeros8 = jnp.zeros((n_pad, 128), jnp.float32)
    pdeg = _make_deg_kernel(n_pad, kd)(col_deg, ones8, zeros8)

    y_split, dinv = _tc_scale_call(x, W_gcn, pdeg, n_pad, bn)

    ka = e // (NS * CH_AGG)
    row3 = row.reshape(NS * ka, CH_AGG)
    col3 = col.reshape(NS * ka, CH_AGG)
    h_agg = _make_agg_kernel(n_pad, ka)(row3, col3, y_split)

    return _tc_heads_call(h_agg, dinv, b_gcn, W_mu, b_mu, W_ls, b_ls, n, bn)
